# baseline (speedup 1.0000x reference)
import functools

import jax
import jax.numpy as jnp
from jax import lax
from jax.experimental import pallas as pl
from jax.experimental.pallas import tpu as pltpu
from jax.experimental.pallas import tpu_sc as plsc

NW = 32
LANE = 128
K = 8


def _take_kernel(total, ch):
    mesh = plsc.VectorSubcoreMesh(core_axis_name="c", subcore_axis_name="s")

    @functools.partial(
        pl.kernel,
        mesh=mesh,
        out_type=jax.ShapeDtypeStruct((NW, ch, LANE), jnp.float32),
        scratch_types=[
            pltpu.VMEM((ch, LANE), jnp.int32),
            pltpu.VMEM((ch, LANE), jnp.float32),
            pltpu.SemaphoreType.DMA,
        ],
    )
    def k(x_hbm, idx_hbm, out_hbm, idx_v, out_v, sem):
        wid = lax.axis_index("s") * 2 + lax.axis_index("c")
        pltpu.sync_copy(idx_hbm.at[wid], idx_v)

        def group(g, _):
            base = g * K
            for b in range(K):
                pltpu.make_async_copy(
                    x_hbm.at[idx_v.at[base + b]], out_v.at[base + b], sem
                ).start()
            for b in range(K):
                pltpu.make_async_copy(
                    x_hbm.at[idx_v.at[base + b]], out_v.at[base + b], sem
                ).wait()
            return 0

        lax.fori_loop(0, ch // K, group, 0, unroll=False)
        pltpu.sync_copy(out_v, out_hbm.at[wid])

    return k


def kernel(x, index):
    n_out = index.shape[0] * index.shape[1]
    ch = n_out // (NW * LANE)
    idx = index.reshape(-1).astype(jnp.int32).reshape(NW, ch, LANE)
    out = _take_kernel(n_out, ch)(x, idx)
    return out.reshape(index.shape)


# speedup vs baseline: 1.0863x; 1.0863x over previous
import functools

import jax
import jax.numpy as jnp
from jax import lax
from jax.experimental import pallas as pl
from jax.experimental.pallas import tpu as pltpu
from jax.experimental.pallas import tpu_sc as plsc

NW = 32
LANE = 128
K = 16


def _take_kernel(total, ch):
    mesh = plsc.VectorSubcoreMesh(core_axis_name="c", subcore_axis_name="s")

    @functools.partial(
        pl.kernel,
        mesh=mesh,
        out_type=jax.ShapeDtypeStruct((NW, ch, LANE), jnp.float32),
        scratch_types=[
            pltpu.VMEM((ch, LANE), jnp.int32),
            pltpu.VMEM((ch, LANE), jnp.float32),
            pltpu.SemaphoreType.DMA,
        ],
    )
    def k(x_hbm, idx_hbm, out_hbm, idx_v, out_v, sem):
        wid = lax.axis_index("s") * 2 + lax.axis_index("c")
        pltpu.sync_copy(idx_hbm.at[wid], idx_v)

        def body(j, _):
            pltpu.make_async_copy(
                x_hbm.at[idx_v.at[j]], out_v.at[j], sem
            ).start()

            @pl.when(j >= K)
            def _w():
                pltpu.make_async_copy(
                    x_hbm.at[idx_v.at[j - K]], out_v.at[j - K], sem
                ).wait()

            return 0

        lax.fori_loop(0, ch, body, 0, unroll=False)

        def tail(j, _):
            pltpu.make_async_copy(
                x_hbm.at[idx_v.at[ch - K + j]], out_v.at[ch - K + j], sem
            ).wait()
            return 0

        lax.fori_loop(0, K, tail, 0, unroll=False)
        pltpu.sync_copy(out_v, out_hbm.at[wid])

    return k


def kernel(x, index):
    n_out = index.shape[0] * index.shape[1]
    ch = n_out // (NW * LANE)
    idx = index.reshape(-1).astype(jnp.int32).reshape(NW, ch, LANE)
    out = _take_kernel(n_out, ch)(x, idx)
    return out.reshape(index.shape)


# speedup vs baseline: 1.7908x; 1.6486x over previous
import functools

import jax
import jax.numpy as jnp
from jax import lax
from jax.experimental import pallas as pl
from jax.experimental.pallas import tpu as pltpu
from jax.experimental.pallas import tpu_sc as plsc

NW = 32
LANE = 128
K = 16
TABLE = 1_000_000
CHUNK = 62_496
SUB = 15_624
BCH = 80


def _take_kernel(ch):
    nb = ch // BCH
    mesh = plsc.VectorSubcoreMesh(core_axis_name="c", subcore_axis_name="s")

    @functools.partial(
        pl.kernel,
        mesh=mesh,
        out_type=jax.ShapeDtypeStruct((NW, ch, LANE), jnp.float32),
        scratch_types=[
            pltpu.VMEM((2, BCH, LANE), jnp.int32),
            pltpu.VMEM((2, BCH, LANE), jnp.float32),
            pltpu.VMEM((SUB,), jnp.float32),
            pltpu.VMEM_SHARED((TABLE,), jnp.float32),
            pltpu.SemaphoreType.DMA,
            pltpu.SemaphoreType.DMA,
            pltpu.SemaphoreType.DMA,
        ],
    )
    def k(x_hbm, idx_hbm, out_hbm, idx2, out2, bounce, table_sh, sem,
          sem_idx, sem_out):
        cid = lax.axis_index("c")
        sid = lax.axis_index("s")
        wid = sid * 2 + cid

        def idx_load(blk, slot):
            return pltpu.make_async_copy(
                idx_hbm.at[wid, pl.ds(blk * BCH, BCH)], idx2.at[slot],
                sem_idx)

        def out_store(blk, slot):
            return pltpu.make_async_copy(
                out2.at[slot], out_hbm.at[wid, pl.ds(blk * BCH, BCH)],
                sem_out)

        idx_load(0, 0).start()

        off = sid * CHUNK
        for p in range(CHUNK // SUB):
            s = off + p * SUB
            pltpu.sync_copy(x_hbm.at[pl.ds(s, SUB)], bounce)
            pltpu.sync_copy(bounce, table_sh.at[pl.ds(s, SUB)])

        @pl.when(sid == 15)
        def _tail():
            pltpu.sync_copy(x_hbm.at[pl.ds(16 * CHUNK, 64)],
                            bounce.at[pl.ds(0, 64)])
            pltpu.sync_copy(bounce.at[pl.ds(0, 64)],
                            table_sh.at[pl.ds(16 * CHUNK, 64)])

        plsc.subcore_barrier()

        for blk in range(nb):
            slot = blk % 2
            if blk + 1 < nb:
                idx_load(blk + 1, 1 - slot).start()
            idx_load(blk, slot).wait()
            if blk >= 2:
                out_store(blk - 2, slot).wait()

            def body(j, _):
                pltpu.make_async_copy(
                    table_sh.at[idx2.at[slot, j]], out2.at[slot, j], sem
                ).start()

                @pl.when(j >= K)
                def _w():
                    pltpu.make_async_copy(
                        table_sh.at[idx2.at[slot, j - K]],
                        out2.at[slot, j - K], sem
                    ).wait()

                return 0

            lax.fori_loop(0, BCH, body, 0, unroll=False)

            def tail(j, _):
                pltpu.make_async_copy(
                    table_sh.at[idx2.at[slot, BCH - K + j]],
                    out2.at[slot, BCH - K + j], sem
                ).wait()
                return 0

            lax.fori_loop(0, K, tail, 0, unroll=False)
            out_store(blk, slot).start()

        out_store(nb - 2, nb % 2).wait()
        out_store(nb - 1, 1 - nb % 2).wait()

    return k


def kernel(x, index):
    n_out = index.shape[0] * index.shape[1]
    ch = n_out // (NW * LANE)
    idx = index.reshape(-1).astype(jnp.int32).reshape(NW, ch, LANE)
    out = _take_kernel(ch)(x, idx)
    return out.reshape(index.shape)


# speedup vs baseline: 1.8819x; 1.0509x over previous
import functools

import jax
import jax.numpy as jnp
from jax import lax
from jax.experimental import pallas as pl
from jax.experimental.pallas import tpu as pltpu
from jax.experimental.pallas import tpu_sc as plsc

NW = 32
TABLE = 1_000_000
CHUNK = 62_496
SUB = 15_624
BN = 10_240


def _take_kernel(per_w):
    nb = per_w // BN
    mesh = plsc.VectorSubcoreMesh(core_axis_name="c", subcore_axis_name="s")

    @functools.partial(
        pl.kernel,
        mesh=mesh,
        out_type=jax.ShapeDtypeStruct((NW, per_w), jnp.float32),
        scratch_types=[
            pltpu.VMEM((BN,), jnp.int32),
            pltpu.VMEM((BN,), jnp.int32),
            pltpu.VMEM((BN,), jnp.float32),
            pltpu.VMEM((BN,), jnp.float32),
            pltpu.VMEM((SUB,), jnp.float32),
            pltpu.VMEM_SHARED((TABLE,), jnp.float32),
            pltpu.SemaphoreType.DMA,
            pltpu.SemaphoreType.DMA,
            pltpu.SemaphoreType.DMA,
        ],
    )
    def k(x_hbm, idx_hbm, out_hbm, idx_a, idx_b, out_a, out_b, bounce,
          table_sh, sem, sem_idx, sem_out):
        cid = lax.axis_index("c")
        sid = lax.axis_index("s")
        wid = sid * 2 + cid
        idx_bufs = (idx_a, idx_b)
        out_bufs = (out_a, out_b)

        def idx_load(blk, slot):
            return pltpu.make_async_copy(
                idx_hbm.at[wid, pl.ds(blk * BN, BN)], idx_bufs[slot],
                sem_idx)

        def out_store(blk, slot):
            return pltpu.make_async_copy(
                out_bufs[slot], out_hbm.at[wid, pl.ds(blk * BN, BN)],
                sem_out)

        def gather(slot):
            return pltpu.make_async_copy(
                table_sh.at[idx_bufs[slot]], out_bufs[slot], sem)

        idx_load(0, 0).start()

        off = sid * CHUNK
        for p in range(CHUNK // SUB):
            s = off + p * SUB
            pltpu.sync_copy(x_hbm.at[pl.ds(s, SUB)], bounce)
            pltpu.sync_copy(bounce, table_sh.at[pl.ds(s, SUB)])

        @pl.when(sid == 15)
        def _tail():
            pltpu.sync_copy(x_hbm.at[pl.ds(16 * CHUNK, 64)],
                            bounce.at[pl.ds(0, 64)])
            pltpu.sync_copy(bounce.at[pl.ds(0, 64)],
                            table_sh.at[pl.ds(16 * CHUNK, 64)])

        plsc.subcore_barrier()

        for blk in range(nb):
            slot = blk % 2
            if blk >= 1:
                gather(1 - slot).wait()
                out_store(blk - 1, 1 - slot).start()
            if blk + 1 < nb:
                idx_load(blk + 1, 1 - slot).start()
            idx_load(blk, slot).wait()
            if blk >= 2:
                out_store(blk - 2, slot).wait()
            gather(slot).start()

        gather((nb - 1) % 2).wait()
        out_store(nb - 1, (nb - 1) % 2).start()
        out_store(nb - 2, nb % 2).wait()
        out_store(nb - 1, (nb - 1) % 2).wait()

    return k


def kernel(x, index):
    n_out = index.shape[0] * index.shape[1]
    per_w = n_out // NW
    idx = index.reshape(-1).astype(jnp.int32).reshape(NW, per_w)
    out = _take_kernel(per_w)(x, idx)
    return out.reshape(index.shape)
